# baseline (speedup 1.0000x reference)
import functools

import jax
import jax.numpy as jnp
from jax import lax
from jax.experimental import pallas as pl
from jax.experimental.pallas import tpu as pltpu
from jax.experimental.pallas import tpu_sc as plsc

N_IN = 500000
N_OUT = 1000000
D = 32
NC = 2
NS = 16
NW = NC * NS
CHUNK = 128
SRC = 72
DSTW = CHUNK * D
NFULL = N_OUT // CHUNK
NEXTRA = NFULL % NW
NBASE = NFULL // NW
TAIL = N_OUT - NFULL * CHUNK
TAIL_BASE = NFULL * CHUNK
TAIL_W = 4

_mesh = plsc.VectorSubcoreMesh(core_axis_name="c", subcore_axis_name="s")


def _smap(r):
    return (r + ((r >> 1) & 1)) >> 1


@functools.partial(
    pl.kernel,
    mesh=_mesh,
    out_type=jax.ShapeDtypeStruct((N_OUT * D,), jnp.float32),
    scratch_types=[
        pltpu.VMEM((SRC, D), jnp.float32),
        pltpu.VMEM((SRC, D), jnp.float32),
        pltpu.VMEM((DSTW,), jnp.float32),
        pltpu.VMEM((DSTW,), jnp.float32),
        pltpu.SemaphoreType.DMA,
        pltpu.SemaphoreType.DMA,
        pltpu.SemaphoreType.DMA,
        pltpu.SemaphoreType.DMA,
    ],
)
def _stretch(data_hbm, out_hbm, src0, src1, dst0, dst1, rs0, rs1, ws0, ws1):
    wid = lax.axis_index("s") * NC + lax.axis_index("c")
    count = NBASE + jnp.where(wid < NEXTRA, 1, 0)

    def cidx(i):
        return wid + i * NW

    def fire_read(src, rsem, i):
        pltpu.async_copy(data_hbm.at[pl.ds(cidx(i) * (CHUNK // 2), SRC)],
                         src, rsem)

    def wait_read(src, rsem):
        pltpu.make_async_copy(data_hbm.at[pl.ds(0, SRC)], src, rsem).wait()

    def rearrange(src, dst, nrows, cap):
        for r in range(nrows):
            s = min(_smap(r), cap)
            for h in range(0, D, 16):
                dst[pl.ds(r * D + h, 16)] = src[s, pl.ds(h, 16)]

    def fire_write(dst, wsem, i):
        pltpu.async_copy(dst, out_hbm.at[pl.ds(cidx(i) * DSTW, DSTW)], wsem)

    def wait_write(dst, wsem):
        pltpu.make_async_copy(dst, out_hbm.at[pl.ds(0, DSTW)], wsem).wait()

    fire_read(src0, rs0, 0)
    fire_read(src1, rs1, 1)

    def step(src, dst, rsem, wsem, i, first):
        wait_read(src, rsem)

        @pl.when(jnp.logical_not(first))
        def _():
            wait_write(dst, wsem)

        rearrange(src, dst, CHUNK, SRC - 1)
        fire_write(dst, wsem, i)

        @pl.when(i + 2 < count)
        def _():
            fire_read(src, rsem, i + 2)

    def body(p, carry):
        i0, i1 = 2 * p, 2 * p + 1

        @pl.when(i0 < count)
        def _():
            step(src0, dst0, rs0, ws0, i0, p == 0)

        @pl.when(i1 < count)
        def _():
            step(src1, dst1, rs1, ws1, i1, p == 0)

        return carry

    lax.fori_loop(0, (NBASE + 2) // 2, body, 0)

    wait_write(dst0, ws0)
    wait_write(dst1, ws1)

    @pl.when(wid == TAIL_W)
    def _():
        pltpu.async_copy(data_hbm.at[pl.ds(TAIL_BASE // 2, TAIL // 2)],
                         src0.at[pl.ds(0, TAIL // 2)], rs0).wait()
        rearrange(src0, dst0, TAIL, TAIL // 2 - 1)
        pltpu.sync_copy(dst0.at[pl.ds(0, TAIL * D)],
                        out_hbm.at[pl.ds(TAIL_BASE * D, TAIL * D)])


def kernel(data):
    flat = _stretch(data)
    one = (data[0, 0] - data[0, 0]) + jnp.float32(1.0)
    return flat.reshape(N_OUT, D) * one


# speedup vs baseline: 7.8448x; 7.8448x over previous
import functools

import numpy as np

import jax
import jax.numpy as jnp
from jax import lax
from jax.experimental import pallas as pl
from jax.experimental.pallas import tpu as pltpu
from jax.experimental.pallas import tpu_sc as plsc

CIN = 500000
COUT = 1000000
D = 32
NC = 2
NS = 16
RG = 8
NRG = D // RG
NSTRIPE = 8
ITEM = 1024
HALF = ITEM // 2
SRCW = 640
NITEM = 976
K_PER = NITEM // NSTRIPE
FIN_OUT_BASE = NITEM * ITEM
FIN_OUT_LEN = 512
FIN_SRC_BASE = FIN_OUT_BASE // 2
FIN_SRC_MAIN = 256
TAIL_IN_BASE = FIN_SRC_BASE + FIN_SRC_MAIN
TAIL_IN = CIN - TAIL_IN_BASE
OUT_PATCH = COUT - (FIN_OUT_BASE + FIN_OUT_LEN)

_mesh = plsc.VectorSubcoreMesh(core_axis_name="c", subcore_axis_name="s")


def _smap(r):
    return (r + ((r >> 1) & 1)) >> 1


@functools.partial(
    pl.kernel,
    mesh=_mesh,
    out_type=jax.ShapeDtypeStruct((D, COUT), jnp.float32),
    scratch_types=[
        pltpu.VMEM((RG, SRCW), jnp.float32),
        pltpu.VMEM((RG, SRCW), jnp.float32),
        pltpu.VMEM((RG, ITEM), jnp.float32),
        pltpu.VMEM((RG, ITEM), jnp.float32),
        pltpu.VMEM((RG, TAIL_IN), jnp.float32),
        pltpu.SemaphoreType.DMA,
        pltpu.SemaphoreType.DMA,
        pltpu.SemaphoreType.DMA,
        pltpu.SemaphoreType.DMA,
    ],
)
def _stretch(dT, tailT, outT, src0, src1, dst0, dst1, tail_v,
             rs0, rs1, ws0, ws1):
    wid = lax.axis_index("s") * NC + lax.axis_index("c")
    rg = wid & (NRG - 1)
    stripe = wid >> 2
    r0 = rg * RG

    lane = lax.iota(jnp.int32, 16)
    pv = (lane + ((lane >> 1) & 1)) >> 1

    gdn = lax.GatherDimensionNumbers(
        offset_dims=(), collapsed_slice_dims=(0,), start_index_map=(0,))

    def dup16(vec):
        return lax.gather(vec, pv[:, None], gdn, slice_sizes=(1,),
                          mode=lax.GatherScatterMode.PROMISE_IN_BOUNDS)

    def fire_read(srcb, sem, k):
        i = stripe + NSTRIPE * k
        pltpu.async_copy(dT.at[pl.ds(r0, RG), pl.ds(i * HALF, SRCW)],
                         srcb, sem)

    def wait_read(srcb, sem):
        pltpu.make_async_copy(dT.at[pl.ds(0, RG), pl.ds(0, SRCW)],
                              srcb, sem).wait()

    def compute(srcb, dstb, ngroups):
        for r in range(RG):
            for g in range(ngroups):
                a = srcb[r, pl.ds(16 * g, 16)]
                b = srcb[r, pl.ds(16 * g + 8, 16)]
                dstb[r, pl.ds(32 * g, 16)] = dup16(a)
                dstb[r, pl.ds(32 * g + 16, 16)] = dup16(b)

    def fire_write(dstb, sem, k):
        i = stripe + NSTRIPE * k
        pltpu.async_copy(dstb, outT.at[pl.ds(r0, RG), pl.ds(i * ITEM, ITEM)],
                         sem)

    def wait_write(dstb, sem):
        pltpu.make_async_copy(dstb, outT.at[pl.ds(0, RG), pl.ds(0, ITEM)],
                              sem).wait()

    fire_read(src0, rs0, 0)
    fire_read(src1, rs1, 1)

    def step(srcb, dstb, rsem, wsem, k, first):
        wait_read(srcb, rsem)

        @pl.when(jnp.logical_not(first))
        def _():
            wait_write(dstb, wsem)

        compute(srcb, dstb, ITEM // 32)
        fire_write(dstb, wsem, k)

        @pl.when(k + 2 < K_PER)
        def _():
            fire_read(srcb, rsem, k + 2)

    def body(p, carry):
        step(src0, dst0, rs0, ws0, 2 * p, p == 0)
        step(src1, dst1, rs1, ws1, 2 * p + 1, p == 0)
        return carry

    lax.fori_loop(0, K_PER // 2, body, 0)

    wait_write(dst0, ws0)
    wait_write(dst1, ws1)

    @pl.when(stripe == NSTRIPE - 1)
    def _():
        pltpu.async_copy(
            dT.at[pl.ds(r0, RG), pl.ds(FIN_SRC_BASE, FIN_SRC_MAIN)],
            src0.at[pl.ds(0, RG), pl.ds(0, FIN_SRC_MAIN)], rs0).wait()
        pltpu.async_copy(tailT.at[pl.ds(r0, RG)], tail_v, rs0).wait()
        for r in range(RG):
            for h in range(0, TAIL_IN, 16):
                src0[r, pl.ds(FIN_SRC_MAIN + h, 16)] = tail_v[r, pl.ds(h, 16)]
        compute(src0, dst0, FIN_OUT_LEN // 32)
        pltpu.sync_copy(
            dst0.at[pl.ds(0, RG), pl.ds(0, FIN_OUT_LEN)],
            outT.at[pl.ds(r0, RG), pl.ds(FIN_OUT_BASE, FIN_OUT_LEN)])


def kernel(data):
    out = _stretch(data.T, data[TAIL_IN_BASE:, :].T).T
    j = np.arange(COUT - OUT_PATCH, COUT)
    tail_idx = jnp.asarray(np.minimum(_smap(j), CIN - 1), jnp.int32)
    patch = jnp.take(data, tail_idx, axis=0)
    return lax.dynamic_update_slice(out, patch, (COUT - OUT_PATCH, 0))


# speedup vs baseline: 9.4073x; 1.1992x over previous
import functools

import numpy as np

import jax
import jax.numpy as jnp
from jax import lax
from jax.experimental import pallas as pl
from jax.experimental.pallas import tpu as pltpu
from jax.experimental.pallas import tpu_sc as plsc

CIN = 500000
COUT = 1000000
D = 32
NC = 2
NS = 16
RG = 8
NRG = D // RG
NSTRIPE = 8
ITEM = 2048
HALF = ITEM // 2
SRCW = 1152
NITEM = 999424 // ITEM
K_PER = NITEM // NSTRIPE
FIN_OUT_BASE = NITEM * ITEM
FIN_OUT_LEN = 512
FIN_SRC_BASE = FIN_OUT_BASE // 2
FIN_SRC_MAIN = 256
TAIL_IN_BASE = FIN_SRC_BASE + FIN_SRC_MAIN
TAIL_IN = CIN - TAIL_IN_BASE
OUT_PATCH = COUT - (FIN_OUT_BASE + FIN_OUT_LEN)

_mesh = plsc.VectorSubcoreMesh(core_axis_name="c", subcore_axis_name="s")


def _smap(r):
    return (r + ((r >> 1) & 1)) >> 1


@functools.partial(
    pl.kernel,
    mesh=_mesh,
    out_type=jax.ShapeDtypeStruct((D, COUT), jnp.float32),
    scratch_types=[
        pltpu.VMEM((RG, SRCW), jnp.float32),
        pltpu.VMEM((RG, SRCW), jnp.float32),
        pltpu.VMEM((RG, ITEM), jnp.float32),
        pltpu.VMEM((RG, ITEM), jnp.float32),
        pltpu.VMEM((RG, TAIL_IN), jnp.float32),
        pltpu.SemaphoreType.DMA,
        pltpu.SemaphoreType.DMA,
        pltpu.SemaphoreType.DMA,
        pltpu.SemaphoreType.DMA,
    ],
)
def _stretch(dT, tailT, outT, src0, src1, dst0, dst1, tail_v,
             rs0, rs1, ws0, ws1):
    wid = lax.axis_index("s") * NC + lax.axis_index("c")
    rg = wid & (NRG - 1)
    stripe = wid >> 2
    r0 = rg * RG

    lane = lax.iota(jnp.int32, 16)
    pv = (lane + ((lane >> 1) & 1)) >> 1

    gdn = lax.GatherDimensionNumbers(
        offset_dims=(), collapsed_slice_dims=(0,), start_index_map=(0,))

    def dup16(vec):
        return lax.gather(vec, pv[:, None], gdn, slice_sizes=(1,),
                          mode=lax.GatherScatterMode.PROMISE_IN_BOUNDS)

    def fire_read(srcb, sem, k):
        i = stripe + NSTRIPE * k
        pltpu.async_copy(dT.at[pl.ds(r0, RG), pl.ds(i * HALF, SRCW)],
                         srcb, sem)

    def wait_read(srcb, sem):
        pltpu.make_async_copy(dT.at[pl.ds(0, RG), pl.ds(0, SRCW)],
                              srcb, sem).wait()

    def compute(srcb, dstb, ngroups):
        def crow(r, carry):
            for g in range(ngroups):
                a = srcb[r, pl.ds(16 * g, 16)]
                b = srcb[r, pl.ds(16 * g + 8, 16)]
                dstb[r, pl.ds(32 * g, 16)] = dup16(a)
                dstb[r, pl.ds(32 * g + 16, 16)] = dup16(b)
            return carry
        lax.fori_loop(0, RG, crow, 0)

    def fire_write(dstb, sem, k):
        i = stripe + NSTRIPE * k
        pltpu.async_copy(dstb, outT.at[pl.ds(r0, RG), pl.ds(i * ITEM, ITEM)],
                         sem)

    def wait_write(dstb, sem):
        pltpu.make_async_copy(dstb, outT.at[pl.ds(0, RG), pl.ds(0, ITEM)],
                              sem).wait()

    fire_read(src0, rs0, 0)
    fire_read(src1, rs1, 1)

    def step(srcb, dstb, rsem, wsem, k, first):
        wait_read(srcb, rsem)

        @pl.when(jnp.logical_not(first))
        def _():
            wait_write(dstb, wsem)

        compute(srcb, dstb, ITEM // 32)
        fire_write(dstb, wsem, k)

        @pl.when(k + 2 < K_PER)
        def _():
            fire_read(srcb, rsem, k + 2)

    def body(p, carry):
        step(src0, dst0, rs0, ws0, 2 * p, p == 0)

        @pl.when(2 * p + 1 < K_PER)
        def _():
            step(src1, dst1, rs1, ws1, 2 * p + 1, p == 0)

        return carry

    lax.fori_loop(0, (K_PER + 1) // 2, body, 0)

    wait_write(dst0, ws0)
    wait_write(dst1, ws1)

    @pl.when(stripe == NSTRIPE - 1)
    def _():
        pltpu.async_copy(
            dT.at[pl.ds(r0, RG), pl.ds(FIN_SRC_BASE, FIN_SRC_MAIN)],
            src0.at[pl.ds(0, RG), pl.ds(0, FIN_SRC_MAIN)], rs0).wait()
        pltpu.async_copy(tailT.at[pl.ds(r0, RG)], tail_v, rs0).wait()
        for r in range(RG):
            for h in range(0, TAIL_IN, 16):
                src0[r, pl.ds(FIN_SRC_MAIN + h, 16)] = tail_v[r, pl.ds(h, 16)]
        compute(src0, dst0, FIN_OUT_LEN // 32)
        pltpu.sync_copy(
            dst0.at[pl.ds(0, RG), pl.ds(0, FIN_OUT_LEN)],
            outT.at[pl.ds(r0, RG), pl.ds(FIN_OUT_BASE, FIN_OUT_LEN)])


def kernel(data):
    out = _stretch(data.T, data[TAIL_IN_BASE:, :].T).T
    j = np.arange(COUT - OUT_PATCH, COUT)
    tail_idx = jnp.asarray(np.minimum(_smap(j), CIN - 1), jnp.int32)
    patch = jnp.take(data, tail_idx, axis=0)
    return lax.dynamic_update_slice(out, patch, (COUT - OUT_PATCH, 0))


# speedup vs baseline: 9.6245x; 1.0231x over previous
import functools

import numpy as np

import jax
import jax.numpy as jnp
from jax import lax
from jax.experimental import pallas as pl
from jax.experimental.pallas import tpu as pltpu
from jax.experimental.pallas import tpu_sc as plsc

CIN = 500000
COUT = 1000000
D = 32
NC = 2
NS = 16
RG = 8
NRG = D // RG
NSTRIPE = 8
ITEM = 4096
HALF = ITEM // 2
SRCW = 2176
NITEM = 999424 // ITEM

FIN_OUT_BASE = NITEM * ITEM
FIN_OUT_LEN = 512
FIN_SRC_BASE = FIN_OUT_BASE // 2
FIN_SRC_MAIN = 256
TAIL_IN_BASE = FIN_SRC_BASE + FIN_SRC_MAIN
TAIL_IN = CIN - TAIL_IN_BASE
OUT_PATCH = COUT - (FIN_OUT_BASE + FIN_OUT_LEN)

_mesh = plsc.VectorSubcoreMesh(core_axis_name="c", subcore_axis_name="s")


def _smap(r):
    return (r + ((r >> 1) & 1)) >> 1


@functools.partial(
    pl.kernel,
    mesh=_mesh,
    out_type=jax.ShapeDtypeStruct((D, COUT), jnp.float32),
    scratch_types=[
        pltpu.VMEM((RG, SRCW), jnp.float32),
        pltpu.VMEM((RG, SRCW), jnp.float32),
        pltpu.VMEM((RG, ITEM), jnp.float32),
        pltpu.VMEM((RG, ITEM), jnp.float32),
        pltpu.VMEM((RG, TAIL_IN), jnp.float32),
        pltpu.SemaphoreType.DMA,
        pltpu.SemaphoreType.DMA,
        pltpu.SemaphoreType.DMA,
        pltpu.SemaphoreType.DMA,
    ],
)
def _stretch(dT, tailT, outT, src0, src1, dst0, dst1, tail_v,
             rs0, rs1, ws0, ws1):
    wid = lax.axis_index("s") * NC + lax.axis_index("c")
    rg = wid & (NRG - 1)
    stripe = wid >> 2
    r0 = rg * RG
    kper = (NITEM // NSTRIPE) + jnp.where(stripe < NITEM % NSTRIPE, 1, 0)

    lane = lax.iota(jnp.int32, 16)
    pv = (lane + ((lane >> 1) & 1)) >> 1

    gdn = lax.GatherDimensionNumbers(
        offset_dims=(), collapsed_slice_dims=(0,), start_index_map=(0,))

    def dup16(vec):
        return lax.gather(vec, pv[:, None], gdn, slice_sizes=(1,),
                          mode=lax.GatherScatterMode.PROMISE_IN_BOUNDS)

    def fire_read(srcb, sem, k):
        i = stripe + NSTRIPE * k
        pltpu.async_copy(dT.at[pl.ds(r0, RG), pl.ds(i * HALF, SRCW)],
                         srcb, sem)

    def wait_read(srcb, sem):
        pltpu.make_async_copy(dT.at[pl.ds(0, RG), pl.ds(0, SRCW)],
                              srcb, sem).wait()

    def compute(srcb, dstb, ngroups):
        def crow(r, carry):
            for g in range(ngroups):
                a = srcb[r, pl.ds(16 * g, 16)]
                b = srcb[r, pl.ds(16 * g + 8, 16)]
                dstb[r, pl.ds(32 * g, 16)] = dup16(a)
                dstb[r, pl.ds(32 * g + 16, 16)] = dup16(b)
            return carry
        lax.fori_loop(0, RG, crow, 0)

    def fire_write(dstb, sem, k):
        i = stripe + NSTRIPE * k
        pltpu.async_copy(dstb, outT.at[pl.ds(r0, RG), pl.ds(i * ITEM, ITEM)],
                         sem)

    def wait_write(dstb, sem):
        pltpu.make_async_copy(dstb, outT.at[pl.ds(0, RG), pl.ds(0, ITEM)],
                              sem).wait()

    fire_read(src0, rs0, 0)
    fire_read(src1, rs1, 1)

    def step(srcb, dstb, rsem, wsem, k, first):
        wait_read(srcb, rsem)

        @pl.when(jnp.logical_not(first))
        def _():
            wait_write(dstb, wsem)

        compute(srcb, dstb, ITEM // 32)
        fire_write(dstb, wsem, k)

        @pl.when(k + 2 < kper)
        def _():
            fire_read(srcb, rsem, k + 2)

    def body(p, carry):
        step(src0, dst0, rs0, ws0, 2 * p, p == 0)

        @pl.when(2 * p + 1 < kper)
        def _():
            step(src1, dst1, rs1, ws1, 2 * p + 1, p == 0)

        return carry

    lax.fori_loop(0, (kper + 1) // 2, body, 0)

    wait_write(dst0, ws0)
    wait_write(dst1, ws1)

    @pl.when(stripe == NSTRIPE - 1)
    def _():
        pltpu.async_copy(
            dT.at[pl.ds(r0, RG), pl.ds(FIN_SRC_BASE, FIN_SRC_MAIN)],
            src0.at[pl.ds(0, RG), pl.ds(0, FIN_SRC_MAIN)], rs0).wait()
        pltpu.async_copy(tailT.at[pl.ds(r0, RG)], tail_v, rs0).wait()
        for r in range(RG):
            for h in range(0, TAIL_IN, 16):
                src0[r, pl.ds(FIN_SRC_MAIN + h, 16)] = tail_v[r, pl.ds(h, 16)]
        compute(src0, dst0, FIN_OUT_LEN // 32)
        pltpu.sync_copy(
            dst0.at[pl.ds(0, RG), pl.ds(0, FIN_OUT_LEN)],
            outT.at[pl.ds(r0, RG), pl.ds(FIN_OUT_BASE, FIN_OUT_LEN)])


def kernel(data):
    out = _stretch(data.T, data[TAIL_IN_BASE:, :].T).T
    j = np.arange(COUT - OUT_PATCH, COUT)
    tail_idx = jnp.asarray(np.minimum(_smap(j), CIN - 1), jnp.int32)
    patch = jnp.take(data, tail_idx, axis=0)
    return lax.dynamic_update_slice(out, patch, (COUT - OUT_PATCH, 0))
